# Initial kernel scaffold; baseline (speedup 1.0000x reference)
import jax
import jax.numpy as jnp
from jax.experimental import pallas as pl


def kernel(x, W, iInd, jInd):
    raise NotImplementedError("write your pallas kernel here")


# speedup vs baseline: 24.6675x; 24.6675x over previous
import jax
import jax.numpy as jnp
from jax import lax
from jax.experimental import pallas as pl
from jax.experimental.pallas import tpu as pltpu
from jax.experimental.pallas import tpu_sc as plsc

N_NODES = 10000
N_EDGES = 320000
C = 128
NC = 2
NS = 16
NW = NC * NS
K = 128
NCHUNK = N_EDGES // K
NCHUNK_CORE = NCHUNK // NC
NR = 624
NR_LAST_EXTRA = N_NODES - NS * NR
SHIFT = 14


def _transpose_body(x_ref, o_ref):
    o_ref[...] = x_ref[...].T


def _transpose(x2d):
    return pl.pallas_call(
        _transpose_body,
        out_shape=jax.ShapeDtypeStruct((N_NODES, C), jnp.float32),
    )(x2d)


def _pack_body(i_ref, j_ref, o_ref):
    o_ref[...] = (i_ref[...] << SHIFT) | j_ref[...]


def _pack(iInd, jInd):
    return pl.pallas_call(
        _pack_body,
        out_shape=jax.ShapeDtypeStruct((NCHUNK, K), jnp.int32),
    )(iInd.reshape(NCHUNK, K), jInd.reshape(NCHUNK, K)).reshape(
        NC, N_EDGES // NC)


def _edge_kernel_body(xT, Wh, pk, om_hbm, osum_hbm, w_v, pk_v, idx_i, idx_j,
                      rows, s_v, om, sem):
    cid = lax.axis_index("c")
    sid = lax.axis_index("s")

    zero16 = jnp.zeros((16,), jnp.float32)

    def zrow(r, carry):
        for v in range(C // 16):
            rows[r, pl.ds(v * 16, 16)] = zero16
        return carry

    lax.fori_loop(0, K, zrow, 0)

    def zs(r, carry):
        s_v[pl.ds(r * 16, 16)] = zero16
        return carry

    lax.fori_loop(0, N_NODES // 16, zs, 0)

    nbase = sid * NR
    def zacc(t, carry):
        pltpu.sync_copy(rows.at[pl.ds(0, 104)],
                        om.at[pl.ds(nbase + t * 104, 104)])
        return carry
    lax.fori_loop(0, NR // 104, zacc, 0)

    @pl.when(sid == NS - 1)
    def _zero_tail():
        pltpu.sync_copy(rows.at[pl.ds(0, NR_LAST_EXTRA)],
                        om.at[pl.ds(NS * NR, NR_LAST_EXTRA)])

    pltpu.sync_copy(Wh, w_v)
    plsc.subcore_barrier()

    nch = (NCHUNK_CORE - sid + NS - 1) // NS

    def chunk_body(t, carry):
        ch = sid + t * NS
        base = ch * K
        pltpu.sync_copy(pk.at[cid, pl.ds(base, K)], pk_v)

        def unpack(g, c2):
            v = pk_v[pl.ds(g * 16, 16)]
            idx_i[pl.ds(g * 16, 16)] = v >> SHIFT
            idx_j[pl.ds(g * 16, 16)] = v & ((1 << SHIFT) - 1)
            return c2

        lax.fori_loop(0, K // 16, unpack, 0)
        pltpu.async_copy(xT.at[idx_j], rows, sem).wait()

        def grp(g, c2):
            vi = idx_i[pl.ds(g * 16, 16)]
            vj = idx_j[pl.ds(g * 16, 16)]
            wi = plsc.load_gather(w_v, [vi])
            wj = plsc.load_gather(w_v, [vj])
            cv = wi * (wi + wj)
            plsc.addupdate_scatter(s_v, [vi], cv)
            for k in range(16):
                e = g * 16 + k
                cs = cv[k]
                for v in range(C // 16):
                    rows[e, pl.ds(v * 16, 16)] = (
                        rows[e, pl.ds(v * 16, 16)] * cs)
            return c2

        lax.fori_loop(0, K // 16, grp, 0)
        pltpu.sync_copy(rows, om.at[idx_i], add=True)
        return carry

    lax.fori_loop(0, nch, chunk_body, 0)
    plsc.subcore_barrier()

    pltpu.sync_copy(om.at[pl.ds(nbase, NR)],
                    om_hbm.at[cid, pl.ds(nbase, NR)])

    @pl.when(sid == NS - 1)
    def _tail():
        pltpu.sync_copy(om.at[pl.ds(NS * NR, NR_LAST_EXTRA)],
                        om_hbm.at[cid, pl.ds(NS * NR, NR_LAST_EXTRA)])

    wid = cid * NS + sid
    pltpu.sync_copy(s_v, osum_hbm.at[pl.ds(wid * N_NODES, N_NODES)])


def _edge_scatter(xT, W, packed):
    mesh = plsc.VectorSubcoreMesh(core_axis_name="c", subcore_axis_name="s",
                                  num_cores=NC, num_subcores=NS)
    f = pl.kernel(
        _edge_kernel_body,
        out_type=(jax.ShapeDtypeStruct((NC, N_NODES, C), jnp.float32),
                  jax.ShapeDtypeStruct((NW * N_NODES,), jnp.float32)),
        mesh=mesh,
        compiler_params=pltpu.CompilerParams(needs_layout_passes=False),
        scratch_types=[
            pltpu.VMEM((N_NODES,), jnp.float32),
            pltpu.VMEM((K,), jnp.int32),
            pltpu.VMEM((K,), jnp.int32),
            pltpu.VMEM((K,), jnp.int32),
            pltpu.VMEM((K, C), jnp.float32),
            pltpu.VMEM((N_NODES,), jnp.float32),
            pltpu.VMEM_SHARED((N_NODES, C), jnp.float32),
            pltpu.SemaphoreType.DMA,
        ],
    )
    return f(xT, W, packed)


def _combine_body(x_ref, am_ref, as_ref, o_ref):
    s = jnp.sum(as_ref[...], axis=0, keepdims=True)
    a = am_ref[0] + am_ref[1]
    o_ref[...] = x_ref[...] * s - a.T


def _combine(x2d, om, osum):
    return pl.pallas_call(
        _combine_body,
        out_shape=jax.ShapeDtypeStruct((C, N_NODES), jnp.float32),
    )(x2d, om, osum.reshape(NW, N_NODES))


def kernel(x, W, iInd, jInd):
    x2d = x[0]
    xT = _transpose(x2d)
    packed = _pack(iInd.astype(jnp.int32), jInd.astype(jnp.int32))
    om, osum = _edge_scatter(xT, W, packed)
    out2d = _combine(x2d, om, osum)
    return out2d[None]
